# baseline (speedup 1.0000x reference)
import functools

import jax
import jax.numpy as jnp
from jax.experimental import pallas as pl
from jax.experimental.pallas import tpu as pltpu

EPS = 1e-12
Q = 784
D = 512
K = 16384
KT = 1024
NT = K // KT


def _patchcore_kernel(q_ref, m_ref, minval_ref, s_ref,
                      allm_ref, qaug_ref, vec_ref, ming_ref):
    t = pl.program_id(0)
    nt = pl.num_programs(0)

    @pl.when(t == 0)
    def _build_qaug():
        qaug_ref[:, 0:D] = -q_ref[...]
        qaug_ref[:, D:D + 1] = jnp.ones((Q, 1), jnp.float32)

    m = m_ref[...]
    allm_ref[pl.ds(t * KT, KT), 0:D] = m
    allm_ref[pl.ds(t * KT, KT), D:D + 1] = 0.5 * jnp.sum(
        m * m, axis=1, keepdims=True)
    h = jax.lax.dot_general(
        qaug_ref[...], allm_ref[pl.ds(t * KT, KT), :],
        (((1,), (1,)), ((), ())),
        preferred_element_type=jnp.float32)
    rowmin = jnp.min(h, axis=1, keepdims=True)

    @pl.when(t == 0)
    def _init():
        ming_ref[...] = rowmin

    @pl.when(t > 0)
    def _update():
        ming_ref[...] = jnp.minimum(rowmin, ming_ref[...])

    @pl.when(t == nt - 1)
    def _finalize():
        q = q_ref[...]
        q2 = jnp.sum(q * q, axis=1, keepdims=True)
        mv2 = jnp.maximum(2.0 * ming_ref[...] + q2, 0.0)
        minval_ref[...] = jnp.sqrt(mv2 + EPS)
        smax = jnp.max(mv2)
        rows = jax.lax.broadcasted_iota(jnp.int32, mv2.shape, 0)
        sidx = jnp.min(jnp.where(mv2 == smax, rows, Q))
        s_star = jnp.sqrt(smax + EPS)

        mt = q_ref[pl.ds(sidx, 1), :]
        t2 = jnp.sum(mt * mt)
        vec_ref[0:1, 0:D] = -mt
        vec_ref[0:1, D:D + 1] = jnp.ones((1, 1), jnp.float32)
        ct = jax.lax.dot_general(
            vec_ref[...], allm_ref[...], (((1,), (1,)), ((), ())),
            preferred_element_type=jnp.float32)
        wt = jnp.maximum(2.0 * ct + t2, 0.0)
        flat = jax.lax.broadcasted_iota(jnp.int32, wt.shape, 1)
        star = jnp.min(jnp.where(wt == jnp.min(wt), flat, K))
        mstar = allm_ref[pl.ds(star, 1), 0:D]
        s2 = jnp.sum(mstar * mstar)
        vec_ref[0:1, 0:D] = -mstar
        cs = jax.lax.dot_general(
            vec_ref[...], allm_ref[...], (((1,), (1,)), ((), ())),
            preferred_element_type=jnp.float32)
        ws = jnp.maximum(2.0 * cs + s2, 0.0)
        acc = 0.0
        for _ in range(3):
            mn = jnp.min(ws)
            idx = jnp.min(jnp.where(ws == mn, flat, K))
            dj2 = jnp.min(jnp.where(flat == idx, wt, jnp.inf))
            acc = acc + jnp.exp(jnp.sqrt(dj2 + EPS))
            ws = jnp.where(flat == idx, jnp.inf, ws)
        s_ref[0, 0] = (1.0 - jnp.exp(s_star) / acc) * s_star


@functools.partial(jax.jit, static_argnums=())
def kernel(queries, memory):
    minval, s = pl.pallas_call(
        _patchcore_kernel,
        grid=(NT,),
        in_specs=[
            pl.BlockSpec((Q, D), lambda t: (0, 0)),
            pl.BlockSpec((KT, D), lambda t: (t, 0)),
        ],
        out_specs=[
            pl.BlockSpec((Q, 1), lambda t: (0, 0)),
            pl.BlockSpec(memory_space=pltpu.SMEM),
        ],
        out_shape=[
            jax.ShapeDtypeStruct((Q, 1), jnp.float32),
            jax.ShapeDtypeStruct((1, 1), jnp.float32),
        ],
        scratch_shapes=[
            pltpu.VMEM((K, D + 1), jnp.float32),
            pltpu.VMEM((Q, D + 1), jnp.float32),
            pltpu.VMEM((1, D + 1), jnp.float32),
            pltpu.VMEM((Q, 1), jnp.float32),
        ],
    )(queries, memory)

    return (s[0, 0], minval.reshape(Q))


# speedup vs baseline: 1.0521x; 1.0521x over previous
import functools

import jax
import jax.numpy as jnp
from jax.experimental import pallas as pl
from jax.experimental.pallas import tpu as pltpu

EPS = 1e-12
Q = 784
D = 512
K = 16384
KT = 1024
NT = K // KT


def _patchcore_kernel(q_ref, m_ref, minval_ref, s_ref,
                      allm_ref, m2h_ref, ming_ref):
    t = pl.program_id(0)
    nt = pl.num_programs(0)

    m = m_ref[...]
    allm_ref[pl.ds(t * KT, KT), :] = m
    qm = jax.lax.dot_general(
        q_ref[...], m, (((1,), (1,)), ((), ())),
        preferred_element_type=jnp.float32)
    m2h = 0.5 * jax.lax.dot_general(
        jnp.ones((1, D), jnp.float32), m * m, (((1,), (1,)), ((), ())),
        preferred_element_type=jnp.float32)
    m2h_ref[0:1, pl.ds(t * KT, KT)] = m2h
    h = m2h - qm
    rowmin = jnp.min(h, axis=1, keepdims=True)

    @pl.when(t == 0)
    def _init():
        ming_ref[...] = rowmin

    @pl.when(t > 0)
    def _update():
        ming_ref[...] = jnp.minimum(rowmin, ming_ref[...])

    @pl.when(t == nt - 1)
    def _finalize():
        q = q_ref[...]
        q2 = jnp.sum(q * q, axis=1, keepdims=True)
        mv2 = jnp.maximum(2.0 * ming_ref[...] + q2, 0.0)
        minval_ref[...] = jnp.sqrt(mv2 + EPS)
        smax = jnp.max(mv2)
        rows = jax.lax.broadcasted_iota(jnp.int32, mv2.shape, 0)
        sidx = jnp.min(jnp.where(mv2 == smax, rows, Q))
        s_star = jnp.sqrt(smax + EPS)

        mt = q_ref[pl.ds(sidx, 1), :]
        t2 = jnp.sum(mt * mt)
        m2h_row = m2h_ref[...]
        ct = jax.lax.dot_general(
            mt, allm_ref[...], (((1,), (1,)), ((), ())),
            preferred_element_type=jnp.float32)
        wt = jnp.maximum(2.0 * (m2h_row - ct) + t2, 0.0)
        flat = jax.lax.broadcasted_iota(jnp.int32, wt.shape, 1)
        star = jnp.min(jnp.where(wt == jnp.min(wt), flat, K))
        mstar = allm_ref[pl.ds(star, 1), :]
        s2 = jnp.sum(mstar * mstar)
        cs = jax.lax.dot_general(
            mstar, allm_ref[...], (((1,), (1,)), ((), ())),
            preferred_element_type=jnp.float32)
        ws = jnp.maximum(2.0 * (m2h_row - cs) + s2, 0.0)
        acc = 0.0
        for _ in range(3):
            mn = jnp.min(ws)
            idx = jnp.min(jnp.where(ws == mn, flat, K))
            dj2 = jnp.min(jnp.where(flat == idx, wt, jnp.inf))
            acc = acc + jnp.exp(jnp.sqrt(dj2 + EPS))
            ws = jnp.where(flat == idx, jnp.inf, ws)
        s_ref[0, 0] = (1.0 - jnp.exp(s_star) / acc) * s_star


@functools.partial(jax.jit, static_argnums=())
def kernel(queries, memory):
    minval, s = pl.pallas_call(
        _patchcore_kernel,
        grid=(NT,),
        in_specs=[
            pl.BlockSpec((Q, D), lambda t: (0, 0)),
            pl.BlockSpec((KT, D), lambda t: (t, 0)),
        ],
        out_specs=[
            pl.BlockSpec((Q, 1), lambda t: (0, 0)),
            pl.BlockSpec(memory_space=pltpu.SMEM),
        ],
        out_shape=[
            jax.ShapeDtypeStruct((Q, 1), jnp.float32),
            jax.ShapeDtypeStruct((1, 1), jnp.float32),
        ],
        scratch_shapes=[
            pltpu.VMEM((K, D), jnp.float32),
            pltpu.VMEM((1, K), jnp.float32),
            pltpu.VMEM((Q, 1), jnp.float32),
        ],
    )(queries, memory)

    return (s[0, 0], minval.reshape(Q))
